# baseline (speedup 1.0000x reference)
import functools

import jax
import jax.numpy as jnp
from jax import lax
from jax.experimental import pallas as pl
from jax.experimental.pallas import tpu as pltpu
from jax.experimental.pallas import tpu_sc as plsc

F32 = jnp.float32
BF16 = jnp.bfloat16
_NC, _NS = 2, 16
_NW = _NC * _NS
_LAT = 64
_HALF = 32

_BE = 2000
_BN = 2000


def _ln(h, g, b):
    m = jnp.mean(h, axis=-1, keepdims=True)
    d = h - m
    v = jnp.mean(d * d, axis=-1, keepdims=True)
    return d * lax.rsqrt(v + 1e-5) * g + b


def _dot(a, b):
    return jnp.dot(a, b, preferred_element_type=F32)


def _node_enc_body(x_ref, v_ref, rho_ref, pt_ref, wx, wv, wr, wt, b1, w2, b2,
                   w3, b3, g, bln, out_ref, tab_ref):
    x = x_ref[...]
    vel = v_ref[...]
    rho = rho_ref[...]
    pt = pt_ref[...]
    nb = x.shape[0]
    oh = (pt == lax.broadcasted_iota(jnp.int32, (nb, wt.shape[0]), 1)
          ).astype(F32)
    h = (_dot(x, wx[...]) + _dot(vel, wv[...]) + rho * wr[...]
         + _dot(oh, wt[...]) + b1[...])
    h = jnp.maximum(h, 0.0)
    h = jnp.maximum(_dot(h, w2[...]) + b2[...], 0.0)
    h = _dot(h, w3[...]) + b3[...]
    res = _ln(h, g[...], bln[...])
    out_ref[...] = res
    tab_ref[...] = res.astype(BF16)


def _mlp3(x, w1, b1, w2, b2, w3, b3):
    h = jnp.maximum(_dot(x, w1[...]) + b1[...], 0.0)
    h = jnp.maximum(_dot(h, w2[...]) + b2[...], 0.0)
    return _dot(h, w3[...]) + b3[...]


def _edge_upd_body(first, e_ref, gsd_ref, *ww):
    out_ref = ww[-1]
    ww = ww[:-1]
    if first:
        (ew1, eb1, ew2, eb2, ew3, eb3, eg, ebln) = ww[:8]
        ww = ww[8:]
        e = _ln(_mlp3(e_ref[...], ew1, eb1, ew2, eb2, ew3, eb3),
                eg[...], ebln[...])
    else:
        e = e_ref[...][:, _LAT:]
    (w1, b1, w2, b2, w3, b3, g, bln) = ww
    x = jnp.concatenate([e, gsd_ref[...].astype(F32)], axis=-1)
    u = _ln(_mlp3(x, w1, b1, w2, b2, w3, b3), g[...], bln[...])
    out_ref[...] = jnp.concatenate([u, e + u], axis=-1)


def _node_upd_body(decode, n_ref, agg_ref, wn, wa, b1, w2,
                   b2, w3, b3, g, bln, *rest):
    n = n_ref[...]
    agg = agg_ref[...][:, :_LAT]
    h = _dot(n, wn[...]) + _dot(agg, wa[...]) + b1[...]
    h = jnp.maximum(h, 0.0)
    h = jnp.maximum(_dot(h, w2[...]) + b2[...], 0.0)
    h = _dot(h, w3[...]) + b3[...]
    nn = n + _ln(h, g[...], bln[...])
    if decode:
        dw1, db1, dw2, db2, dw3, db3 = rest[:6]
        out_ref = rest[6]
        h = jnp.maximum(_dot(nn, dw1[...]) + db1[...], 0.0)
        h = jnp.maximum(_dot(h, dw2[...]) + db2[...], 0.0)
        out_ref[...] = _dot(h, dw3[...]) + db3[...]
    else:
        rest[0][...] = nn
        rest[1][...] = nn.astype(BF16)


def _full_spec(arr):
    nd = arr.ndim
    return pl.BlockSpec(arr.shape, lambda i, _nd=nd: (0,) * _nd)


def _tc_call(body, grid, row_specs, weight_arrs, out_specs, out_shapes):
    return pl.pallas_call(
        body,
        grid=(grid,),
        in_specs=row_specs + [_full_spec(w) for w in weight_arrs],
        out_specs=out_specs,
        out_shape=out_shapes,
        compiler_params=pltpu.CompilerParams(
            dimension_semantics=("arbitrary",)),
    )


_ROW = 128
_GK = 7


def _gather_pair(nodes, src2, dst2):
    nrows = src2.shape[0]
    e = nrows * _ROW
    rpw = -(-nrows // _NW)
    nsteps = -(-rpw // _GK)
    rmax = nrows - _GK
    mesh = plsc.VectorSubcoreMesh(core_axis_name="c", subcore_axis_name="s")

    @functools.partial(
        pl.kernel,
        out_type=jax.ShapeDtypeStruct((e, 2 * _LAT), BF16),
        mesh=mesh,
        scratch_types=[pltpu.VMEM((_GK, _ROW), jnp.int32),
                       pltpu.VMEM((_GK * _ROW, _LAT), BF16),
                       pltpu.SemaphoreType.DMA],
        compiler_params=pltpu.CompilerParams(use_tc_tiling_on_sc=False),
    )
    def k(nodes_hbm, src_hbm, dst_hbm, gsd_hbm, idx_v, rows_v, sem):
        wid = lax.axis_index("s") * _NC + lax.axis_index("c")
        base = wid * rpw

        def pass_(idx_hbm, col0):
            def step(t, carry):
                row0 = jnp.minimum(base + t * _GK, rmax)
                pltpu.sync_copy(idx_hbm.at[pl.ds(row0, _GK)], idx_v)
                descs = [
                    pltpu.async_copy(
                        nodes_hbm.at[idx_v.at[i]],
                        rows_v.at[pl.ds(i * _ROW, _ROW)], sem)
                    for i in range(_GK)
                ]
                for d in descs:
                    d.wait()
                pltpu.sync_copy(
                    rows_v,
                    gsd_hbm.at[pl.ds(row0 * _ROW, _GK * _ROW),
                               pl.ds(col0, _LAT)])
                return carry
            lax.fori_loop(0, nsteps, step, 0)

        pass_(src_hbm, 0)
        pass_(dst_hbm, _LAT)

    return k(nodes, src2, dst2)


_SK = 5


def _segment_sum_packed(upk, dst2, n, zeros_chunk):
    nrows = dst2.shape[0]
    rpt_full = -(-nrows // (_NS * _SK)) * _SK
    rpt_last = nrows - rpt_full * (_NS - 1)
    assert rpt_full % _SK == 0 and rpt_last % _SK == 0 and rpt_last > 0
    zr = n // _NS
    zrows = zeros_chunk.shape[0]
    nz = zr // zrows
    mesh = plsc.VectorSubcoreMesh(core_axis_name="c", subcore_axis_name="s")

    @functools.partial(
        pl.kernel,
        out_type=jax.ShapeDtypeStruct((n, 2 * _LAT), F32),
        mesh=mesh,
        scratch_types=[pltpu.VMEM_SHARED((n, _HALF), F32),
                       pltpu.VMEM((_SK, _ROW), jnp.int32),
                       pltpu.VMEM((_SK * _ROW, _HALF), F32),
                       pltpu.SemaphoreType.DMA],
        compiler_params=pltpu.CompilerParams(use_tc_tiling_on_sc=False),
    )
    def k(upk_hbm, dst_hbm, z_hbm, agg_hbm, slab, idx_v, val_v, sem):
        c = lax.axis_index("c")
        s = lax.axis_index("s")
        col0 = c * _HALF
        pltpu.sync_copy(z_hbm, val_v.at[pl.ds(0, zrows)])
        for kk in range(nz):
            pltpu.sync_copy(val_v.at[pl.ds(0, zrows)],
                            slab.at[pl.ds(s * zr + kk * zrows, zrows)])
        plsc.subcore_barrier()

        base = s * rpt_full
        nsteps = jnp.where(s == _NS - 1, rpt_last // _SK, rpt_full // _SK)

        def step(t, carry):
            row0 = base + t * _SK
            pltpu.sync_copy(dst_hbm.at[pl.ds(row0, _SK)], idx_v)
            pltpu.sync_copy(
                upk_hbm.at[pl.ds(row0 * _ROW, _SK * _ROW),
                           pl.ds(col0, _HALF)], val_v)
            descs = [
                pltpu.async_copy(
                    val_v.at[pl.ds(i * _ROW, _ROW)],
                    slab.at[idx_v.at[i]], sem, add=True)
                for i in range(_SK)
            ]
            for d in descs:
                d.wait()
            return carry
        lax.fori_loop(0, nsteps, step, 0)

        plsc.subcore_barrier()

        for kk in range(nz):
            r = s * zr + kk * zrows
            pltpu.sync_copy(slab.at[pl.ds(r, zrows)],
                            val_v.at[pl.ds(0, zrows)])
            pltpu.sync_copy(val_v.at[pl.ds(0, zrows)],
                            agg_hbm.at[pl.ds(r, zrows), pl.ds(col0, _HALF)])

    return k(upk, dst2, zeros_chunk)


def kernel(x, v, rho, particle_type, edge_index, edge_features, params):
    n = x.shape[0]
    e = edge_features.shape[0]
    src2 = edge_index[0].astype(jnp.int32).reshape(e // _ROW, _ROW)
    dst2 = edge_index[1].astype(jnp.int32).reshape(e // _ROW, _ROW)

    def rowspec(b, width=None):
        if width is None:
            return pl.BlockSpec((b,), lambda i: (i,))
        return pl.BlockSpec((b, width), lambda i: (i, 0))

    (w1, b1), (w2, b2), (w3, b3) = params['enc_node']
    g, bln = params['enc_node_ln']
    wx, wv = w1[0:3], w1[3:6]
    wr = w1[6:7]
    wt = params['type_emb'] @ w1[7:23]
    nodes, ntab = _tc_call(
        _node_enc_body, n // _BN,
        [rowspec(_BN, 3), rowspec(_BN, 3), rowspec(_BN, 1), rowspec(_BN, 1)],
        [wx, wv, wr, wt, b1, w2, b2, w3, b3, g, bln],
        [rowspec(_BN, _LAT), rowspec(_BN, _LAT)],
        [jax.ShapeDtypeStruct((n, _LAT), F32),
         jax.ShapeDtypeStruct((n, _LAT), BF16)],
    )(x, v, rho[:, None], particle_type.astype(jnp.int32)[:, None],
      wx, wv, wr, wt, b1, w2, b2, w3, b3, g, bln)

    zeros_chunk = jnp.zeros((n // _NS // 5, _HALF), F32)
    (encw1, encb1), (encw2, encb2), (encw3, encb3) = params['enc_edge']
    encg, encbln = params['enc_edge_ln']
    enc_weights = [encw1, encb1, encw2, encb2, encw3, encb3, encg, encbln]
    ein = edge_features.shape[1]

    nproc = len(params['proc'])
    upk = edge_features
    for si, p in enumerate(params['proc']):
        first = si == 0
        last = si == nproc - 1

        gsd = _gather_pair(ntab, src2, dst2)

        (w1, b1), (w2, b2), (w3, b3) = p['edge_mlp']
        g, bln = p['edge_ln']
        weights = (enc_weights if first else []) + \
            [w1, b1, w2, b2, w3, b3, g, bln]
        upk = _tc_call(
            functools.partial(_edge_upd_body, first), e // _BE,
            [rowspec(_BE, ein if first else 2 * _LAT),
             rowspec(_BE, 2 * _LAT)],
            weights,
            rowspec(_BE, 2 * _LAT), jax.ShapeDtypeStruct((e, 2 * _LAT), F32),
        )(upk, gsd, *weights)

        agg = _segment_sum_packed(upk, dst2, n, zeros_chunk)

        (w1, b1), (w2, b2), (w3, b3) = p['node_mlp']
        g, bln = p['node_ln']
        wn, wa = w1[0:_LAT], w1[_LAT:]
        weights = [wn, wa, b1, w2, b2, w3, b3, g, bln]
        if last:
            (dw1, db1), (dw2, db2), (dw3, db3) = params['dec']
            weights += [dw1, db1, dw2, db2, dw3, db3]
            out_spec = rowspec(_BN, 1)
            out_shape = jax.ShapeDtypeStruct((n, 1), F32)
        else:
            out_spec = [rowspec(_BN, _LAT), rowspec(_BN, _LAT)]
            out_shape = [jax.ShapeDtypeStruct((n, _LAT), F32),
                         jax.ShapeDtypeStruct((n, _LAT), BF16)]
        res = _tc_call(
            functools.partial(_node_upd_body, last), n // _BN,
            [rowspec(_BN, _LAT), rowspec(_BN, 2 * _LAT)],
            weights, out_spec, out_shape,
        )(nodes, agg, *weights)
        if not last:
            nodes, ntab = res
        else:
            nodes = res

    return nodes


# speedup vs baseline: 1.8562x; 1.8562x over previous
import functools

import jax
import jax.numpy as jnp
from jax import lax
from jax.experimental import pallas as pl
from jax.experimental.pallas import tpu as pltpu
from jax.experimental.pallas import tpu_sc as plsc

F32 = jnp.float32
BF16 = jnp.bfloat16
_NC, _NS = 2, 16
_NW = _NC * _NS
_LAT = 64
_HALF = 32

_BE = 6400
_BN = 2000


def _ln(h, g, b):
    m = jnp.mean(h, axis=-1, keepdims=True)
    d = h - m
    v = jnp.mean(d * d, axis=-1, keepdims=True)
    return d * lax.rsqrt(v + 1e-5) * g + b


def _dot(a, b):
    return jnp.dot(a, b, preferred_element_type=F32)


def _node_enc_body(x_ref, v_ref, rho_ref, pt_ref, wx, wv, wr, wt, b1, w2, b2,
                   w3, b3, g, bln, out_ref):
    x = x_ref[...]
    vel = v_ref[...]
    rho = rho_ref[...]
    pt = pt_ref[...]
    nb = x.shape[0]
    oh = (pt == lax.broadcasted_iota(jnp.int32, (nb, wt.shape[0]), 1)
          ).astype(F32)
    h = (_dot(x, wx[...]) + _dot(vel, wv[...]) + rho * wr[...]
         + _dot(oh, wt[...]) + b1[...])
    h = jnp.maximum(h, 0.0)
    h = jnp.maximum(_dot(h, w2[...]) + b2[...], 0.0)
    h = _dot(h, w3[...]) + b3[...]
    out_ref[...] = _ln(h, g[...], bln[...])


def _mlp3(x, w1, b1, w2, b2, w3, b3):
    h = jnp.maximum(_dot(x, w1[...]) + b1[...], 0.0)
    h = jnp.maximum(_dot(h, w2[...]) + b2[...], 0.0)
    return _dot(h, w3[...]) + b3[...]


def _edge_upd_body(first, e_ref, gsd_ref, *ww):
    out_ref = ww[-1]
    ww = ww[:-1]
    if first:
        (ew1, eb1, ew2, eb2, ew3, eb3, eg, ebln) = ww[:8]
        ww = ww[8:]
        h = lax.dot_general(e_ref[...], ew1[...], (((0,), (0,)), ((), ())),
                            preferred_element_type=F32)
        h = jnp.maximum(h + eb1[...], 0.0)
        h = jnp.maximum(_dot(h, ew2[...]) + eb2[...], 0.0)
        h = _dot(h, ew3[...]) + eb3[...]
        e = _ln(h, eg[...], ebln[...])
    else:
        e = e_ref[...][:, _LAT:]
    (w1, b1, w2, b2, w3, b3, g, bln) = ww
    x = jnp.concatenate([e, gsd_ref[...]], axis=-1)
    u = _ln(_mlp3(x, w1, b1, w2, b2, w3, b3), g[...], bln[...])
    out_ref[...] = jnp.concatenate([u, e + u], axis=-1)


def _node_upd_body(decode, n_ref, agg_ref, wn, wa, b1, w2,
                   b2, w3, b3, g, bln, *rest):
    n = n_ref[...]
    agg = agg_ref[...][:, :_LAT]
    h = _dot(n, wn[...]) + _dot(agg, wa[...]) + b1[...]
    h = jnp.maximum(h, 0.0)
    h = jnp.maximum(_dot(h, w2[...]) + b2[...], 0.0)
    h = _dot(h, w3[...]) + b3[...]
    nn = n + _ln(h, g[...], bln[...])
    if decode:
        dw1, db1, dw2, db2, dw3, db3 = rest[:6]
        out_ref = rest[6]
        h = jnp.maximum(_dot(nn, dw1[...]) + db1[...], 0.0)
        h = jnp.maximum(_dot(h, dw2[...]) + db2[...], 0.0)
        out_ref[...] = _dot(h, dw3[...]) + db3[...]
    else:
        rest[0][...] = nn


def _full_spec(arr):
    nd = arr.ndim
    return pl.BlockSpec(arr.shape, lambda i, _nd=nd: (0,) * _nd)


def _tc_call(body, grid, row_specs, weight_arrs, out_specs, out_shapes):
    return pl.pallas_call(
        body,
        grid=(grid,),
        in_specs=row_specs + [_full_spec(w) for w in weight_arrs],
        out_specs=out_specs,
        out_shape=out_shapes,
        compiler_params=pltpu.CompilerParams(
            dimension_semantics=("arbitrary",)),
    )


_ROW = 128
_GK = 7


def _gather_pair(nodes, src2, dst2):
    nrows = src2.shape[0]
    e = nrows * _ROW
    rpw = -(-nrows // _NW)
    nsteps = -(-rpw // _GK)
    rmax = nrows - _GK
    mesh = plsc.VectorSubcoreMesh(core_axis_name="c", subcore_axis_name="s")

    @functools.partial(
        pl.kernel,
        out_type=jax.ShapeDtypeStruct((e, 2 * _LAT), F32),
        mesh=mesh,
        scratch_types=[pltpu.VMEM((_GK, _ROW), jnp.int32),
                       pltpu.VMEM((_GK * _ROW, _LAT), F32),
                       pltpu.SemaphoreType.DMA],
        compiler_params=pltpu.CompilerParams(use_tc_tiling_on_sc=False),
    )
    def k(nodes_hbm, src_hbm, dst_hbm, gsd_hbm, idx_v, rows_v, sem):
        wid = lax.axis_index("s") * _NC + lax.axis_index("c")
        base = wid * rpw

        def pass_(idx_hbm, col0):
            def step(t, carry):
                row0 = jnp.minimum(base + t * _GK, rmax)
                pltpu.sync_copy(idx_hbm.at[pl.ds(row0, _GK)], idx_v)
                descs = [
                    pltpu.async_copy(
                        nodes_hbm.at[idx_v.at[i]],
                        rows_v.at[pl.ds(i * _ROW, _ROW)], sem)
                    for i in range(_GK)
                ]
                for d in descs:
                    d.wait()
                pltpu.sync_copy(
                    rows_v,
                    gsd_hbm.at[pl.ds(row0 * _ROW, _GK * _ROW),
                               pl.ds(col0, _LAT)])
                return carry
            lax.fori_loop(0, nsteps, step, 0)

        pass_(src_hbm, 0)
        pass_(dst_hbm, _LAT)

    return k(nodes, src2, dst2)


_SK = 5


def _segment_sum_packed(upk, dst2, n, zeros_chunk):
    nrows = dst2.shape[0]
    rpt_full = -(-nrows // (_NS * _SK)) * _SK
    rpt_last = nrows - rpt_full * (_NS - 1)
    assert rpt_full % _SK == 0 and rpt_last % _SK == 0 and rpt_last > 0
    zr = n // _NS
    zrows = zeros_chunk.shape[0]
    nz = zr // zrows
    mesh = plsc.VectorSubcoreMesh(core_axis_name="c", subcore_axis_name="s")

    @functools.partial(
        pl.kernel,
        out_type=jax.ShapeDtypeStruct((n, 2 * _LAT), F32),
        mesh=mesh,
        scratch_types=[pltpu.VMEM_SHARED((n, _HALF), F32),
                       pltpu.VMEM((_SK, _ROW), jnp.int32),
                       pltpu.VMEM((_SK * _ROW, _HALF), F32),
                       pltpu.SemaphoreType.DMA],
        compiler_params=pltpu.CompilerParams(use_tc_tiling_on_sc=False),
    )
    def k(upk_hbm, dst_hbm, z_hbm, agg_hbm, slab, idx_v, val_v, sem):
        c = lax.axis_index("c")
        s = lax.axis_index("s")
        col0 = c * _HALF
        pltpu.sync_copy(z_hbm, val_v.at[pl.ds(0, zrows)])
        for kk in range(nz):
            pltpu.sync_copy(val_v.at[pl.ds(0, zrows)],
                            slab.at[pl.ds(s * zr + kk * zrows, zrows)])
        plsc.subcore_barrier()

        base = s * rpt_full
        nsteps = jnp.where(s == _NS - 1, rpt_last // _SK, rpt_full // _SK)

        def step(t, carry):
            row0 = base + t * _SK
            pltpu.sync_copy(dst_hbm.at[pl.ds(row0, _SK)], idx_v)
            pltpu.sync_copy(
                upk_hbm.at[pl.ds(row0 * _ROW, _SK * _ROW),
                           pl.ds(col0, _HALF)], val_v)
            descs = [
                pltpu.async_copy(
                    val_v.at[pl.ds(i * _ROW, _ROW)],
                    slab.at[idx_v.at[i]], sem, add=True)
                for i in range(_SK)
            ]
            for d in descs:
                d.wait()
            return carry
        lax.fori_loop(0, nsteps, step, 0)

        plsc.subcore_barrier()

        for kk in range(nz):
            r = s * zr + kk * zrows
            pltpu.sync_copy(slab.at[pl.ds(r, zrows)],
                            val_v.at[pl.ds(0, zrows)])
            pltpu.sync_copy(val_v.at[pl.ds(0, zrows)],
                            agg_hbm.at[pl.ds(r, zrows), pl.ds(col0, _HALF)])

    return k(upk, dst2, zeros_chunk)


def kernel(x, v, rho, particle_type, edge_index, edge_features, params):
    n = x.shape[0]
    e = edge_features.shape[0]
    src2 = edge_index[0].astype(jnp.int32).reshape(e // _ROW, _ROW)
    dst2 = edge_index[1].astype(jnp.int32).reshape(e // _ROW, _ROW)

    def rowspec(b, width=None):
        if width is None:
            return pl.BlockSpec((b,), lambda i: (i,))
        return pl.BlockSpec((b, width), lambda i: (i, 0))

    (w1, b1), (w2, b2), (w3, b3) = params['enc_node']
    g, bln = params['enc_node_ln']
    wx, wv = w1[0:3], w1[3:6]
    wr = w1[6:7]
    wt = params['type_emb'] @ w1[7:23]
    nodes = _tc_call(
        _node_enc_body, n // _BN,
        [rowspec(_BN, 3), rowspec(_BN, 3), rowspec(_BN, 1), rowspec(_BN, 1)],
        [wx, wv, wr, wt, b1, w2, b2, w3, b3, g, bln],
        rowspec(_BN, _LAT), jax.ShapeDtypeStruct((n, _LAT), F32),
    )(x, v, rho[:, None], particle_type.astype(jnp.int32)[:, None],
      wx, wv, wr, wt, b1, w2, b2, w3, b3, g, bln)

    zeros_chunk = jnp.zeros((n // _NS // 5, _HALF), F32)
    (encw1, encb1), (encw2, encb2), (encw3, encb3) = params['enc_edge']
    encg, encbln = params['enc_edge_ln']
    ein = edge_features.shape[1]
    ef8 = jnp.pad(edge_features.T, ((0, 8 - ein), (0, 0)))
    encw1p = jnp.pad(encw1, ((0, 8 - ein), (0, 0)))
    enc_weights = [encw1p, encb1, encw2, encb2, encw3, encb3, encg, encbln]

    nproc = len(params['proc'])
    upk = ef8
    for si, p in enumerate(params['proc']):
        first = si == 0
        last = si == nproc - 1

        gsd = _gather_pair(nodes, src2, dst2)

        (w1, b1), (w2, b2), (w3, b3) = p['edge_mlp']
        g, bln = p['edge_ln']
        weights = (enc_weights if first else []) + \
            [w1, b1, w2, b2, w3, b3, g, bln]
        espec = (pl.BlockSpec((8, _BE), lambda i: (0, i)) if first
                 else rowspec(_BE, 2 * _LAT))
        upk = _tc_call(
            functools.partial(_edge_upd_body, first), e // _BE,
            [espec, rowspec(_BE, 2 * _LAT)],
            weights,
            rowspec(_BE, 2 * _LAT), jax.ShapeDtypeStruct((e, 2 * _LAT), F32),
        )(upk, gsd, *weights)

        agg = _segment_sum_packed(upk, dst2, n, zeros_chunk)

        (w1, b1), (w2, b2), (w3, b3) = p['node_mlp']
        g, bln = p['node_ln']
        wn, wa = w1[0:_LAT], w1[_LAT:]
        weights = [wn, wa, b1, w2, b2, w3, b3, g, bln]
        if last:
            (dw1, db1), (dw2, db2), (dw3, db3) = params['dec']
            weights += [dw1, db1, dw2, db2, dw3, db3]
            out_spec = rowspec(_BN, 1)
            out_shape = jax.ShapeDtypeStruct((n, 1), F32)
        else:
            out_spec = rowspec(_BN, _LAT)
            out_shape = jax.ShapeDtypeStruct((n, _LAT), F32)
        res = _tc_call(
            functools.partial(_node_upd_body, last), n // _BN,
            [rowspec(_BN, _LAT), rowspec(_BN, 2 * _LAT)],
            weights, out_spec, out_shape,
        )(nodes, agg, *weights)
        nodes = res

    return nodes


# speedup vs baseline: 2.1755x; 1.1721x over previous
import functools

import jax
import jax.numpy as jnp
from jax import lax
from jax.experimental import pallas as pl
from jax.experimental.pallas import tpu as pltpu
from jax.experimental.pallas import tpu_sc as plsc

F32 = jnp.float32
BF16 = jnp.bfloat16
_NC, _NS = 2, 16
_NW = _NC * _NS
_LAT = 64
_HALF = 32

_BE = 6400
_BN = 2000


def _ln(h, g, b):
    m = jnp.mean(h, axis=-1, keepdims=True)
    d = h - m
    v = jnp.mean(d * d, axis=-1, keepdims=True)
    return d * lax.rsqrt(v + 1e-5) * g + b


def _dot(a, b):
    return jnp.dot(a, b, preferred_element_type=F32)


def _node_enc_body(x_ref, v_ref, rho_ref, pt_ref, wx, wv, wr, wt, b1, w2, b2,
                   w3, b3, g, bln, out_ref):
    x = x_ref[...]
    vel = v_ref[...]
    rho = rho_ref[...]
    pt = pt_ref[...]
    nb = x.shape[0]
    oh = (pt == lax.broadcasted_iota(jnp.int32, (nb, wt.shape[0]), 1)
          ).astype(F32)
    h = (_dot(x, wx[...]) + _dot(vel, wv[...]) + rho * wr[...]
         + _dot(oh, wt[...]) + b1[...])
    h = jnp.maximum(h, 0.0)
    h = jnp.maximum(_dot(h, w2[...]) + b2[...], 0.0)
    h = _dot(h, w3[...]) + b3[...]
    out_ref[...] = _ln(h, g[...], bln[...])


def _mlp3(x, w1, b1, w2, b2, w3, b3):
    h = jnp.maximum(_dot(x, w1[...]) + b1[...], 0.0)
    h = jnp.maximum(_dot(h, w2[...]) + b2[...], 0.0)
    return _dot(h, w3[...]) + b3[...]


def _edge_upd_body(first, e_ref, gsd_ref, *ww):
    out_ref = ww[-1]
    ww = ww[:-1]
    if first:
        (ew1, eb1, ew2, eb2, ew3, eb3, eg, ebln) = ww[:8]
        ww = ww[8:]
        h = lax.dot_general(e_ref[...], ew1[...], (((0,), (0,)), ((), ())),
                            preferred_element_type=F32)
        h = jnp.maximum(h + eb1[...], 0.0)
        h = jnp.maximum(_dot(h, ew2[...]) + eb2[...], 0.0)
        h = _dot(h, ew3[...]) + eb3[...]
        e = _ln(h, eg[...], ebln[...])
    else:
        e = e_ref[...][:, _LAT:]
    (w1, b1, w2, b2, w3, b3, g, bln) = ww
    x = jnp.concatenate([e, gsd_ref[...]], axis=-1)
    u = _ln(_mlp3(x, w1, b1, w2, b2, w3, b3), g[...], bln[...])
    out_ref[...] = jnp.concatenate([u, e + u], axis=-1)


def _node_upd_body(decode, n_ref, agg_ref, wn, wa, b1, w2,
                   b2, w3, b3, g, bln, *rest):
    n = n_ref[...]
    agg = agg_ref[...][:, :_LAT]
    h = _dot(n, wn[...]) + _dot(agg, wa[...]) + b1[...]
    h = jnp.maximum(h, 0.0)
    h = jnp.maximum(_dot(h, w2[...]) + b2[...], 0.0)
    h = _dot(h, w3[...]) + b3[...]
    nn = n + _ln(h, g[...], bln[...])
    if decode:
        dw1, db1, dw2, db2, dw3, db3 = rest[:6]
        out_ref = rest[6]
        h = jnp.maximum(_dot(nn, dw1[...]) + db1[...], 0.0)
        h = jnp.maximum(_dot(h, dw2[...]) + db2[...], 0.0)
        out_ref[...] = _dot(h, dw3[...]) + db3[...]
    else:
        rest[0][...] = nn


def _full_spec(arr):
    nd = arr.ndim
    return pl.BlockSpec(arr.shape, lambda i, _nd=nd: (0,) * _nd)


def _tc_call(body, grid, row_specs, weight_arrs, out_specs, out_shapes):
    return pl.pallas_call(
        body,
        grid=(grid,),
        in_specs=row_specs + [_full_spec(w) for w in weight_arrs],
        out_specs=out_specs,
        out_shape=out_shapes,
        compiler_params=pltpu.CompilerParams(
            dimension_semantics=("arbitrary",)),
    )


_ROW = 128
_GK = 7


def _gather_pair(nodes, src2, dst2, row_lo, row_hi):
    nrows = row_hi - row_lo
    e = nrows * _ROW
    rpw = -(-nrows // _NW)
    nsteps = -(-rpw // _GK)
    rmax = row_hi - _GK
    mesh = plsc.VectorSubcoreMesh(core_axis_name="c", subcore_axis_name="s")

    @functools.partial(
        pl.kernel,
        out_type=jax.ShapeDtypeStruct((e, 2 * _LAT), F32),
        mesh=mesh,
        scratch_types=[pltpu.VMEM((_GK, _ROW), jnp.int32),
                       pltpu.VMEM((_GK * _ROW, _LAT), F32),
                       pltpu.SemaphoreType.DMA],
        compiler_params=pltpu.CompilerParams(use_tc_tiling_on_sc=False),
    )
    def k(nodes_hbm, src_hbm, dst_hbm, gsd_hbm, idx_v, rows_v, sem):
        wid = lax.axis_index("s") * _NC + lax.axis_index("c")
        base = row_lo + wid * rpw

        def pass_(idx_hbm, col0):
            def step(t, carry):
                row0 = jnp.minimum(base + t * _GK, rmax)
                pltpu.sync_copy(idx_hbm.at[pl.ds(row0, _GK)], idx_v)
                descs = [
                    pltpu.async_copy(
                        nodes_hbm.at[idx_v.at[i]],
                        rows_v.at[pl.ds(i * _ROW, _ROW)], sem)
                    for i in range(_GK)
                ]
                for d in descs:
                    d.wait()
                pltpu.sync_copy(
                    rows_v,
                    gsd_hbm.at[pl.ds((row0 - row_lo) * _ROW, _GK * _ROW),
                               pl.ds(col0, _LAT)])
                return carry
            lax.fori_loop(0, nsteps, step, 0)

        pass_(src_hbm, 0)
        pass_(dst_hbm, _LAT)

    return k(nodes, src2, dst2)


_SK = 5


def _segment_sum_packed(upk, dst2, row_lo, row_hi, n, init):
    nrows = row_hi - row_lo
    rpt_full = -(-nrows // (_NS * _SK)) * _SK
    rpt_last = nrows - rpt_full * (_NS - 1)
    assert rpt_full % _SK == 0 and rpt_last % _SK == 0 and rpt_last > 0
    zr = n // _NS
    zrows = 625
    assert zr % zrows == 0 and zrows * _HALF <= _SK * _ROW * _HALF
    nz = zr // zrows
    mesh = plsc.VectorSubcoreMesh(core_axis_name="c", subcore_axis_name="s")

    @functools.partial(
        pl.kernel,
        out_type=jax.ShapeDtypeStruct((n, 2 * _LAT), F32),
        mesh=mesh,
        scratch_types=[pltpu.VMEM_SHARED((n, _HALF), F32),
                       pltpu.VMEM((_SK, _ROW), jnp.int32),
                       pltpu.VMEM((_SK * _ROW, _HALF), F32),
                       pltpu.SemaphoreType.DMA],
        compiler_params=pltpu.CompilerParams(use_tc_tiling_on_sc=False),
    )
    def k(upk_hbm, dst_hbm, init_hbm, agg_hbm, slab, idx_v, val_v, sem):
        c = lax.axis_index("c")
        s = lax.axis_index("s")
        col0 = c * _HALF
        for kk in range(nz):
            r = s * zr + kk * zrows
            pltpu.sync_copy(init_hbm.at[pl.ds(r, zrows), pl.ds(col0, _HALF)],
                            val_v.at[pl.ds(0, zrows)])
            pltpu.sync_copy(val_v.at[pl.ds(0, zrows)],
                            slab.at[pl.ds(r, zrows)])
        plsc.subcore_barrier()

        base = row_lo + s * rpt_full
        nsteps = jnp.where(s == _NS - 1, rpt_last // _SK, rpt_full // _SK)

        def step(t, carry):
            row0 = base + t * _SK
            pltpu.sync_copy(dst_hbm.at[pl.ds(row0, _SK)], idx_v)
            pltpu.sync_copy(
                upk_hbm.at[pl.ds((row0 - row_lo) * _ROW, _SK * _ROW),
                           pl.ds(col0, _HALF)], val_v)
            descs = [
                pltpu.async_copy(
                    val_v.at[pl.ds(i * _ROW, _ROW)],
                    slab.at[idx_v.at[i]], sem, add=True)
                for i in range(_SK)
            ]
            for d in descs:
                d.wait()
            return carry
        lax.fori_loop(0, nsteps, step, 0)

        plsc.subcore_barrier()

        for kk in range(nz):
            r = s * zr + kk * zrows
            pltpu.sync_copy(slab.at[pl.ds(r, zrows)],
                            val_v.at[pl.ds(0, zrows)])
            pltpu.sync_copy(val_v.at[pl.ds(0, zrows)],
                            agg_hbm.at[pl.ds(r, zrows), pl.ds(col0, _HALF)])

    return k(upk, dst2, init)


def kernel(x, v, rho, particle_type, edge_index, edge_features, params):
    n = x.shape[0]
    e = edge_features.shape[0]
    src2 = edge_index[0].astype(jnp.int32).reshape(e // _ROW, _ROW)
    dst2 = edge_index[1].astype(jnp.int32).reshape(e // _ROW, _ROW)

    def rowspec(b, width=None):
        if width is None:
            return pl.BlockSpec((b,), lambda i: (i,))
        return pl.BlockSpec((b, width), lambda i: (i, 0))

    (w1, b1), (w2, b2), (w3, b3) = params['enc_node']
    g, bln = params['enc_node_ln']
    wx, wv = w1[0:3], w1[3:6]
    wr = w1[6:7]
    wt = params['type_emb'] @ w1[7:23]
    nodes = _tc_call(
        _node_enc_body, n // _BN,
        [rowspec(_BN, 3), rowspec(_BN, 3), rowspec(_BN, 1), rowspec(_BN, 1)],
        [wx, wv, wr, wt, b1, w2, b2, w3, b3, g, bln],
        rowspec(_BN, _LAT), jax.ShapeDtypeStruct((n, _LAT), F32),
    )(x, v, rho[:, None], particle_type.astype(jnp.int32)[:, None],
      wx, wv, wr, wt, b1, w2, b2, w3, b3, g, bln)

    (encw1, encb1), (encw2, encb2), (encw3, encb3) = params['enc_edge']
    encg, encbln = params['enc_edge_ln']
    ein = edge_features.shape[1]
    ef8 = jnp.pad(edge_features.T, ((0, 8 - ein), (0, 0)))
    encw1p = jnp.pad(encw1, ((0, 8 - ein), (0, 0)))
    enc_weights = [encw1p, encb1, encw2, encb2, encw3, encb3, encg, encbln]

    nrows_all = e // _ROW
    cut = (nrows_all // 2 + (_BE // _ROW) - 1) // (_BE // _ROW) * (_BE // _ROW)
    chunks = [(0, cut), (cut, nrows_all)]
    zeros128 = jnp.zeros((n, 2 * _LAT), F32)

    nproc = len(params['proc'])
    upks = [ef8] * len(chunks)
    for si, p in enumerate(params['proc']):
        first = si == 0
        last = si == nproc - 1

        gsds = [_gather_pair(nodes, src2, dst2, lo, hi) for lo, hi in chunks]

        (w1, b1), (w2, b2), (w3, b3) = p['edge_mlp']
        g, bln = p['edge_ln']
        weights = (enc_weights if first else []) + \
            [w1, b1, w2, b2, w3, b3, g, bln]
        new_upks = []
        for ci, (lo, hi) in enumerate(chunks):
            eh = (hi - lo) * _ROW
            offb = lo * _ROW // _BE
            espec = (pl.BlockSpec((8, _BE), lambda i, o=offb: (0, i + o))
                     if first else rowspec(_BE, 2 * _LAT))
            new_upks.append(_tc_call(
                functools.partial(_edge_upd_body, first), eh // _BE,
                [espec, rowspec(_BE, 2 * _LAT)],
                weights,
                rowspec(_BE, 2 * _LAT),
                jax.ShapeDtypeStruct((eh, 2 * _LAT), F32),
            )(upks[ci], gsds[ci], *weights))
        upks = new_upks

        agg = zeros128
        for ci, (lo, hi) in enumerate(chunks):
            agg = _segment_sum_packed(upks[ci], dst2, lo, hi, n, agg)

        (w1, b1), (w2, b2), (w3, b3) = p['node_mlp']
        g, bln = p['node_ln']
        wn, wa = w1[0:_LAT], w1[_LAT:]
        weights = [wn, wa, b1, w2, b2, w3, b3, g, bln]
        if last:
            (dw1, db1), (dw2, db2), (dw3, db3) = params['dec']
            weights += [dw1, db1, dw2, db2, dw3, db3]
            out_spec = rowspec(_BN, 1)
            out_shape = jax.ShapeDtypeStruct((n, 1), F32)
        else:
            out_spec = rowspec(_BN, _LAT)
            out_shape = jax.ShapeDtypeStruct((n, _LAT), F32)
        res = _tc_call(
            functools.partial(_node_upd_body, last), n // _BN,
            [rowspec(_BN, _LAT), rowspec(_BN, 2 * _LAT)],
            weights, out_spec, out_shape,
        )(nodes, agg, *weights)
        nodes = res

    return nodes
